# baseline (speedup 1.0000x reference)
import functools

import jax
import jax.numpy as jnp
from jax import lax
from jax.experimental import pallas as pl
from jax.experimental.pallas import tpu as pltpu
from jax.experimental.pallas import tpu_sc as plsc

BATCH = 4096
SEQ = 200
DIM = 32
HIDDEN = 8
VOCAB = 1000000
PAD_SEQ = 208
CHUNK = PAD_SEQ // 2
LANES = 16

NC = 2
NS = 16
NW = NC * NS
RPW = BATCH // NW
RIF = 4


def _sc_pool_body(x_hbm, emb_hbm, out_hbm, idx_v, bufs_v, pool_v, *sems):
    wid = lax.axis_index("s") * NC + lax.axis_index("c")
    base = wid * RPW

    pltpu.sync_copy(x_hbm.at[pl.ds(base, RPW)], idx_v)

    for r0 in range(RIF):
        for h in range(2):
            b = r0 * 2 + h
            pltpu.async_copy(emb_hbm.at[idx_v.at[r0, h]], bufs_v.at[b], sems[b])

    neg_inf = jnp.full((LANES,), -jnp.inf, dtype=jnp.float32)

    def group_body(g, carry):
        for r_off in range(RIF):
            r = g * RIF + r_off
            nxt = lax.rem(r + RIF, RPW)
            acc_lo = neg_inf
            acc_hi = neg_inf
            for h in range(2):
                b = r_off * 2 + h
                buf = bufs_v.at[b]
                pltpu.make_async_copy(
                    emb_hbm.at[idx_v.at[r, h]], buf, sems[b]).wait()

                def red(t, acc, buf=buf):
                    lo = jnp.maximum(acc[0], buf[t, pl.ds(0, LANES)])
                    hi = jnp.maximum(acc[1], buf[t, pl.ds(LANES, LANES)])
                    return (lo, hi)

                acc_lo, acc_hi = lax.fori_loop(
                    0, CHUNK, red, (acc_lo, acc_hi), unroll=8)
                pltpu.async_copy(emb_hbm.at[idx_v.at[nxt, h]], buf, sems[b])
            pool_v[r, pl.ds(0, LANES)] = acc_lo
            pool_v[r, pl.ds(LANES, LANES)] = acc_hi
        return carry

    lax.fori_loop(0, RPW // RIF, group_body, 0)

    for r0 in range(RIF):
        for h in range(2):
            b = r0 * 2 + h
            pltpu.make_async_copy(
                emb_hbm.at[idx_v.at[r0, h]], bufs_v.at[b], sems[b]).wait()

    pltpu.sync_copy(pool_v, out_hbm.at[pl.ds(base, RPW)])


_sc_pool = functools.partial(
    pl.kernel,
    out_type=jax.ShapeDtypeStruct((BATCH, DIM), jnp.float32),
    mesh=plsc.VectorSubcoreMesh(core_axis_name="c", subcore_axis_name="s"),
    scratch_types=[
        pltpu.VMEM((RPW, 2, CHUNK), jnp.int32),
        pltpu.VMEM((2 * RIF, CHUNK, DIM), jnp.float32),
        pltpu.VMEM((RPW, DIM), jnp.float32),
    ] + [pltpu.SemaphoreType.DMA] * (2 * RIF),
    compiler_params=pltpu.CompilerParams(use_tc_tiling_on_sc=False),
)(_sc_pool_body)


SLAB = 512
N_SLABS = VOCAB // SLAB
TAIL = VOCAB - (N_SLABS - 1) * SLAB - SLAB


K_MAIN = (N_SLABS - 1) // NW
OROWS = SLAB * DIM // 128


def _sc_detile_body(src_hbm, out_hbm, slab0, slab1, out0, out1,
                    si0, si1, so0, so1):
    wid = lax.axis_index("s") * NC + lax.axis_index("c")
    iota16 = lax.iota(jnp.int32, 16)
    slabs = (slab0, slab1)
    outs = (out0, out1)
    sis = (si0, si1)
    sos = (so0, so1)

    def _in_args(k, slot):
        return (src_hbm.at[:, pl.ds((k * NW + wid) * SLAB, SLAB)],
                slabs[slot].at[:, pl.ds(0, SLAB)], sis[slot])

    def _out_args(k, slot):
        return (outs[slot].at[pl.ds(0, OROWS)],
                out_hbm.at[pl.ds((k * NW + wid) * OROWS, OROWS)], sos[slot])

    def in_copy(k, slot):
        pltpu.async_copy(*_in_args(k, slot))

    def in_wait(k, slot):
        pltpu.make_async_copy(*_in_args(k, slot)).wait()

    def out_copy(k, slot):
        pltpu.async_copy(*_out_args(k, slot))

    def out_wait(k, slot):
        pltpu.make_async_copy(*_out_args(k, slot)).wait()

    def transpose_rows(slab_v, out_v, lo_row, hi_row):
        def row(j, c2):
            q = lax.shift_right_logical(j, 2)
            b = lax.bitwise_and(j, 3)
            jv = jnp.full((16,), j, jnp.int32)
            lo = plsc.load_gather(slab_v, [iota16, jv])
            hi = plsc.load_gather(slab_v, [iota16 + 16, jv])
            out_v[q, pl.ds(b * DIM, LANES)] = lo
            out_v[q, pl.ds(b * DIM + LANES, LANES)] = hi
            return c2
        lax.fori_loop(lo_row, hi_row, row, 0, unroll=4)

    def slot_step(k, slot, g):
        in_wait(k, slot)

        @pl.when(g > 0)
        def _():
            out_wait(k - 2, slot)

        transpose_rows(slabs[slot], outs[slot], 0, SLAB)
        out_copy(k, slot)

    in_copy(0, 0)
    in_copy(1, 1)

    def pair(g, carry):
        slot_step(2 * g, 0, g)
        in_copy(2 * g + 2, 0)
        slot_step(2 * g + 1, 1, g)

        @pl.when(2 * g + 3 < K_MAIN)
        def _():
            in_copy(2 * g + 3, 1)
        return carry

    lax.fori_loop(0, K_MAIN // 2, pair, 0)

    in_wait(K_MAIN - 1, 0)
    out_wait(K_MAIN - 3, 0)
    transpose_rows(slab0, out0, 0, SLAB)
    out_copy(K_MAIN - 1, 0)
    out_wait(K_MAIN - 1, 0)
    out_wait(K_MAIN - 2, 1)

    @pl.when(wid == 0)
    def _():
        s = N_SLABS - 1
        pltpu.async_copy(src_hbm.at[:, pl.ds(s * SLAB, SLAB)],
                         slab0.at[:, pl.ds(0, SLAB)], si0).wait()
        pltpu.async_copy(src_hbm.at[:, pl.ds(VOCAB - TAIL, TAIL)],
                         slab0.at[:, pl.ds(SLAB, TAIL)], si0).wait()
        transpose_rows(slab0, out0, 0, SLAB + TAIL)
        pltpu.async_copy(
            out0, out_hbm.at[pl.ds(s * OROWS, (SLAB + TAIL) * DIM // 128)],
            so0).wait()


_sc_detile = functools.partial(
    pl.kernel,
    out_type=jax.ShapeDtypeStruct((VOCAB * DIM // 128, 128), jnp.float32),
    mesh=plsc.VectorSubcoreMesh(core_axis_name="c", subcore_axis_name="s"),
    scratch_types=[
        pltpu.VMEM((DIM, SLAB + TAIL), jnp.float32),
        pltpu.VMEM((DIM, SLAB), jnp.float32),
        pltpu.VMEM(((SLAB + TAIL) * DIM // 128, 128), jnp.float32),
        pltpu.VMEM((OROWS, 128), jnp.float32),
        pltpu.SemaphoreType.DMA,
        pltpu.SemaphoreType.DMA,
        pltpu.SemaphoreType.DMA,
        pltpu.SemaphoreType.DMA,
    ],
    compiler_params=pltpu.CompilerParams(
        use_tc_tiling_on_sc=True, needs_layout_passes=False),
)(_sc_detile_body)


def _mlp_body(pooled_ref, w1t_ref, b1_ref, w2t_ref, b2_ref, out_ref):
    p = pooled_ref[...]
    h = jnp.dot(p, w1t_ref[...], preferred_element_type=jnp.float32)
    h = jnp.maximum(h + b1_ref[...], 0.0)
    z = jnp.dot(h, w2t_ref[...], preferred_element_type=jnp.float32)
    z = z + b2_ref[...]
    out_ref[...] = 1.0 / (1.0 + jnp.exp(-z))


def kernel(x, emb, W1, b1, W2, b2):
    x = x.astype(jnp.int32)
    x_pad = jnp.concatenate([x, x[:, :PAD_SEQ - SEQ]], axis=1)
    x_pad = x_pad.reshape(BATCH, 2, CHUNK)

    table = _sc_detile(emb.T).reshape(VOCAB, DIM)
    pooled = _sc_pool(x_pad, table)

    out = pl.pallas_call(
        _mlp_body,
        out_shape=jax.ShapeDtypeStruct((BATCH, 1), jnp.float32),
    )(pooled, W1.T, b1.reshape(1, HIDDEN), W2.T, b2.reshape(1, 1))
    return out


# speedup vs baseline: 2.1892x; 2.1892x over previous
import functools

import jax
import jax.numpy as jnp
from jax import lax
from jax.experimental import pallas as pl
from jax.experimental.pallas import tpu as pltpu
from jax.experimental.pallas import tpu_sc as plsc

BATCH = 4096
SEQ = 200
DIM = 32
HIDDEN = 8
VOCAB = 1000000
PAD_SEQ = 208
CHUNK = PAD_SEQ // 2
LANES = 16

NC = 2
NS = 16
NW = NC * NS
RPW = BATCH // NW
RIF = 8

DT_CB = 8192
DT_Q = DT_CB // 4
DT_BLOCKS = (VOCAB + DT_CB - 1) // DT_CB
VOCAB_PAD = DT_BLOCKS * DT_CB


def _sc_pool_body(x_hbm, emb_hbm, w1_hbm, aux_hbm, out_hbm,
                  idx_v, bufs_v, pool_v, w1_v, aux_v, z_v, *sems):
    wid = lax.axis_index("s") * NC + lax.axis_index("c")
    base = wid * RPW

    pltpu.sync_copy(x_hbm.at[pl.ds(base, RPW)], idx_v)
    pltpu.sync_copy(w1_hbm, w1_v)
    pltpu.sync_copy(aux_hbm, aux_v)

    for r0 in range(RIF):
        for h in range(2):
            b = r0 * 2 + h
            pltpu.async_copy(emb_hbm.at[idx_v.at[r0, h]], bufs_v.at[b], sems[b])

    neg_inf = jnp.full((LANES,), -jnp.inf, dtype=jnp.float32)

    def group_body(g, carry):
        for r_off in range(RIF):
            r = g * RIF + r_off
            nxt = lax.rem(r + RIF, RPW)
            acc_lo = neg_inf
            acc_hi = neg_inf
            for h in range(2):
                b = r_off * 2 + h
                buf = bufs_v.at[b]
                pltpu.make_async_copy(
                    emb_hbm.at[idx_v.at[r, h]], buf, sems[b]).wait()

                def red(t, acc, buf=buf):
                    lo = jnp.maximum(acc[0], buf[t, pl.ds(0, LANES)])
                    hi = jnp.maximum(acc[1], buf[t, pl.ds(LANES, LANES)])
                    return (lo, hi)

                acc_lo, acc_hi = lax.fori_loop(
                    0, CHUNK, red, (acc_lo, acc_hi), unroll=8)
                pltpu.async_copy(emb_hbm.at[idx_v.at[nxt, h]], buf, sems[b])
            pool_v[r, pl.ds(0, LANES)] = acc_lo
            pool_v[r, pl.ds(LANES, LANES)] = acc_hi
        return carry

    lax.fori_loop(0, RPW // RIF, group_body, 0)

    for r0 in range(RIF):
        for h in range(2):
            b = r0 * 2 + h
            pltpu.make_async_copy(
                emb_hbm.at[idx_v.at[r0, h]], bufs_v.at[b], sems[b]).wait()

    aux_lo = aux_v[pl.ds(0, LANES)]
    b2 = aux_v[pl.ds(LANES, LANES)][0]
    w1rows = [(w1_v[j, pl.ds(0, LANES)], w1_v[j, pl.ds(LANES, LANES)])
              for j in range(HIDDEN)]
    for c in range(RPW // LANES):
        rows16 = lax.iota(jnp.int32, LANES) + c * LANES
        cols = [plsc.load_gather(pool_v, [rows16, jnp.full((LANES,), d, jnp.int32)])
                for d in range(DIM)]
        z = jnp.full((LANES,), 0.0, jnp.float32) + b2
        for j in range(HIDDEN):
            wlo, whi = w1rows[j]
            acc = jnp.full((LANES,), 0.0, jnp.float32) + aux_lo[j]
            for d in range(LANES):
                acc = acc + cols[d] * wlo[d]
            for d in range(LANES):
                acc = acc + cols[LANES + d] * whi[d]
            z = z + jnp.maximum(acc, 0.0) * aux_lo[HIDDEN + j]
        z_v[pl.ds(c * LANES, LANES)] = 1.0 / (1.0 + jnp.exp(-z))

    pltpu.sync_copy(z_v, out_hbm.at[pl.ds(base, RPW)])


_sc_pool = functools.partial(
    pl.kernel,
    out_type=jax.ShapeDtypeStruct((BATCH,), jnp.float32),
    mesh=plsc.VectorSubcoreMesh(core_axis_name="c", subcore_axis_name="s"),
    scratch_types=[
        pltpu.VMEM((RPW, 2, CHUNK), jnp.int32),
        pltpu.VMEM((2 * RIF, CHUNK, DIM), jnp.float32),
        pltpu.VMEM((RPW, DIM), jnp.float32),
        pltpu.VMEM((HIDDEN, DIM), jnp.float32),
        pltpu.VMEM((DIM,), jnp.float32),
        pltpu.VMEM((RPW,), jnp.float32),
    ] + [pltpu.SemaphoreType.DMA] * (2 * RIF),
    compiler_params=pltpu.CompilerParams(
        use_tc_tiling_on_sc=False, needs_layout_passes=False),
)(_sc_pool_body)


def _detile_body(src_ref, out_ref):
    t = src_ref[...].T
    for a in range(4):
        out_ref[:, DIM * a:DIM * (a + 1)] = t[DT_Q * a:DT_Q * (a + 1), :]


def _detile(emb):
    embt = emb.T
    return pl.pallas_call(
        _detile_body,
        grid=(DT_BLOCKS,),
        in_specs=[pl.BlockSpec((DIM, DT_CB), lambda i: (0, i))],
        out_specs=pl.BlockSpec((DT_CB // 4, 4 * DIM), lambda i: (i, 0)),
        out_shape=jax.ShapeDtypeStruct((VOCAB_PAD * DIM // 128, 128), jnp.float32),
    )(embt)


def _permute_idx(x):
    hi = x & ~(DT_CB - 1)
    return hi + 4 * (x & (DT_Q - 1)) + ((x & (DT_CB - 1)) >> (DT_Q.bit_length() - 1))


def kernel(x, emb, W1, b1, W2, b2):
    x = _permute_idx(x.astype(jnp.int32))
    x_pad = jnp.concatenate([x, x[:, :PAD_SEQ - SEQ]], axis=1)
    x_pad = x_pad.reshape(BATCH, 2, CHUNK)

    table = _detile(emb).reshape(VOCAB_PAD, DIM)
    aux = jnp.concatenate(
        [b1, W2.reshape(HIDDEN), b2, jnp.zeros((DIM - 2 * HIDDEN - 1,), jnp.float32)])
    out = _sc_pool(x_pad, table, W1, aux)
    return out.reshape(BATCH, 1)


# speedup vs baseline: 2.2042x; 1.0068x over previous
import functools

import jax
import jax.numpy as jnp
from jax import lax
from jax.experimental import pallas as pl
from jax.experimental.pallas import tpu as pltpu
from jax.experimental.pallas import tpu_sc as plsc

BATCH = 4096
SEQ = 200
DIM = 32
HIDDEN = 8
VOCAB = 1000000
PAD_SEQ = 208
CHUNK = PAD_SEQ // 2
LANES = 16

NC = 2
NS = 16
NW = NC * NS
RPW = BATCH // NW
RIF = 8

DT_CB = 8192
DT_Q = DT_CB // 4
DT_BLOCKS = (VOCAB + DT_CB - 1) // DT_CB
VOCAB_PAD = DT_BLOCKS * DT_CB


def _sc_pool_body(x_hbm, emb_hbm, out_hbm, idx_v, bufs_v, pool_v, *sems):
    wid = lax.axis_index("s") * NC + lax.axis_index("c")
    base = wid * RPW

    pltpu.sync_copy(x_hbm.at[pl.ds(base, RPW)], idx_v)

    for r0 in range(RIF):
        for h in range(2):
            b = r0 * 2 + h
            pltpu.async_copy(emb_hbm.at[idx_v.at[r0, h]], bufs_v.at[b], sems[b])

    neg_inf = jnp.full((LANES,), -jnp.inf, dtype=jnp.float32)

    def group_body(g, carry):
        for r_off in range(RIF):
            r = g * RIF + r_off
            nxt = lax.rem(r + RIF, RPW)
            acc_lo = neg_inf
            acc_hi = neg_inf
            for h in range(2):
                b = r_off * 2 + h
                buf = bufs_v.at[b]
                pltpu.make_async_copy(
                    emb_hbm.at[idx_v.at[r, h]], buf, sems[b]).wait()

                def red(t, acc, buf=buf):
                    lo = jnp.maximum(acc[0], buf[t, pl.ds(0, LANES)])
                    hi = jnp.maximum(acc[1], buf[t, pl.ds(LANES, LANES)])
                    return (lo, hi)

                acc_lo, acc_hi = lax.fori_loop(
                    0, CHUNK, red, (acc_lo, acc_hi), unroll=8)
                pltpu.async_copy(emb_hbm.at[idx_v.at[nxt, h]], buf, sems[b])
            pool_v[r, pl.ds(0, LANES)] = acc_lo
            pool_v[r, pl.ds(LANES, LANES)] = acc_hi
        return carry

    lax.fori_loop(0, RPW // RIF, group_body, 0)

    for r0 in range(RIF):
        for h in range(2):
            b = r0 * 2 + h
            pltpu.make_async_copy(
                emb_hbm.at[idx_v.at[r0, h]], bufs_v.at[b], sems[b]).wait()

    pltpu.sync_copy(pool_v, out_hbm.at[pl.ds(base, RPW)])


_sc_pool = functools.partial(
    pl.kernel,
    out_type=jax.ShapeDtypeStruct((BATCH, DIM), jnp.float32),
    mesh=plsc.VectorSubcoreMesh(core_axis_name="c", subcore_axis_name="s"),
    scratch_types=[
        pltpu.VMEM((RPW, 2, CHUNK), jnp.int32),
        pltpu.VMEM((2 * RIF, CHUNK, DIM), jnp.float32),
        pltpu.VMEM((RPW, DIM), jnp.float32),
    ] + [pltpu.SemaphoreType.DMA] * (2 * RIF),
    compiler_params=pltpu.CompilerParams(use_tc_tiling_on_sc=False),
)(_sc_pool_body)


def _detile_body(src_ref, out_ref):
    t = src_ref[...].T
    for a in range(4):
        out_ref[:, DIM * a:DIM * (a + 1)] = t[DT_Q * a:DT_Q * (a + 1), :]


def _detile(emb):
    embt = emb.T
    return pl.pallas_call(
        _detile_body,
        grid=(DT_BLOCKS,),
        in_specs=[pl.BlockSpec((DIM, DT_CB), lambda i: (0, i))],
        out_specs=pl.BlockSpec((DT_CB // 4, 4 * DIM), lambda i: (i, 0)),
        out_shape=jax.ShapeDtypeStruct((VOCAB_PAD * DIM // 128, 128), jnp.float32),
    )(embt)


def _permute_idx(x):
    hi = x & ~(DT_CB - 1)
    return hi + 4 * (x & (DT_Q - 1)) + ((x & (DT_CB - 1)) >> (DT_Q.bit_length() - 1))


def _mlp_body(pooled_ref, w1t_ref, b1_ref, w2t_ref, b2_ref, out_ref):
    p = pooled_ref[...]
    h = jnp.dot(p, w1t_ref[...], preferred_element_type=jnp.float32)
    h = jnp.maximum(h + b1_ref[...], 0.0)
    z = jnp.dot(h, w2t_ref[...], preferred_element_type=jnp.float32)
    z = z + b2_ref[...]
    out_ref[...] = 1.0 / (1.0 + jnp.exp(-z))


def kernel(x, emb, W1, b1, W2, b2):
    x = _permute_idx(x.astype(jnp.int32))
    x_pad = jnp.concatenate([x, x[:, :PAD_SEQ - SEQ]], axis=1)
    x_pad = x_pad.reshape(BATCH, 2, CHUNK)

    table = _detile(emb).reshape(VOCAB_PAD, DIM)
    pooled = _sc_pool(x_pad, table)

    out = pl.pallas_call(
        _mlp_body,
        out_shape=jax.ShapeDtypeStruct((BATCH, 1), jnp.float32),
    )(pooled, W1.T, b1.reshape(1, HIDDEN), W2.T, b2.reshape(1, 1))
    return out
